# baseline (speedup 1.0000x reference)
import jax
import jax.numpy as jnp
from jax import lax
from jax.experimental import pallas as pl
from jax.experimental.pallas import tpu as pltpu
from jax.experimental.pallas import tpu_sc as plsc

N = 10000
D = 128
H = 64
NP = 10240
NN = 2 * NP
NC, NS, L = 2, 16, 16
K = 128
NB = 4032
PE = NB * K
NBT = NB // NS
NBC = NB // (NS * NC)
SL2 = NN // NS
DEG2 = (NN // D, D)


def _scdeg_body(einfo, z1, deg_out,
                eb0, eb1, cb0, cb1, ewb0, ewb1, deg_sp,
                sem_e0, sem_e1, sem_s0, sem_s1):
    c = lax.axis_index("c")
    s = lax.axis_index("s")
    sl = pl.ds(s * SL2, SL2)
    pltpu.sync_copy(z1.at[sl], deg_sp.at[sl])
    plsc.subcore_barrier()

    base = (c * NS + s) * NBC
    pltpu.async_copy(einfo.at[base], eb0, sem_e0)
    pltpu.async_copy(einfo.at[base + 1], eb1, sem_e1)

    bufs = ((eb0, cb0, ewb0, sem_e0, sem_s0),
            (eb1, cb1, ewb1, sem_e1, sem_s1))

    def body(g, _):
        for b, (eb, cb, ewb, sem_e, sem_s) in enumerate(bufs):
            i = g * 2 + b
            pltpu.make_async_copy(einfo.at[0], eb, sem_e).wait()

            @pl.when(g > 0)
            def _():
                pltpu.make_async_copy(z1.at[pl.ds(0, K)], ewb, sem_s).wait()

            for j in range(K // L):
                jj = pl.ds(j * L, L)
                cb[jj] = eb[1, jj]
                ewb[jj] = plsc.bitcast(eb[2, jj], jnp.float32)

            @pl.when(i + 2 < NBC)
            def _():
                pltpu.async_copy(einfo.at[base + i + 2], eb, sem_e)

            pltpu.async_copy(ewb, deg_sp.at[cb], sem_s, add=True)
        return 0

    lax.fori_loop(0, NBC // 2, body, 0)
    pltpu.make_async_copy(z1.at[pl.ds(0, K)], ewb0, sem_s0).wait()
    pltpu.make_async_copy(z1.at[pl.ds(0, K)], ewb1, sem_s1).wait()
    plsc.subcore_barrier()
    pltpu.sync_copy(deg_sp.at[sl], deg_out.at[c, sl])


def _sc_deg(einfo, z1):
    f32 = jnp.float32
    i32 = jnp.int32
    kern = pl.kernel(
        _scdeg_body,
        mesh=plsc.VectorSubcoreMesh(core_axis_name="c", subcore_axis_name="s"),
        compiler_params=pltpu.CompilerParams(
            needs_layout_passes=False, use_tc_tiling_on_sc=False),
        out_type=jax.ShapeDtypeStruct((NC, NN), f32),
        scratch_types=[
            pltpu.VMEM((3, K), i32), pltpu.VMEM((3, K), i32),
            pltpu.VMEM((K,), i32), pltpu.VMEM((K,), i32),
            pltpu.VMEM((K,), f32), pltpu.VMEM((K,), f32),
            pltpu.VMEM_SHARED((NN,), f32),
            pltpu.SemaphoreType.DMA, pltpu.SemaphoreType.DMA,
            pltpu.SemaphoreType.DMA, pltpu.SemaphoreType.DMA,
        ],
    )
    return kern(einfo, z1)


def _mm_body(nf, df, nw, dw, dg, xw, dv):
    xw[0] = jnp.dot(
        nf[...], nw[...], preferred_element_type=jnp.float32
    ).astype(jnp.bfloat16)
    xw[1] = jnp.dot(
        df[...], dw[...], preferred_element_type=jnp.float32
    ).astype(jnp.bfloat16)

    @pl.when(pl.program_id(0) == 0)
    def _():
        degsum = dg[0] + dg[1]
        dv[...] = jnp.where(degsum > 0, lax.rsqrt(degsum), 0.0)


def _t1(net_feat, net_W, dag_feat, dag_W, deg):
    MB = 640
    cs = lambda shape: pl.BlockSpec(shape, lambda m: tuple(0 for _ in shape))
    return pl.pallas_call(
        _mm_body,
        grid=(NP // MB,),
        in_specs=[
            pl.BlockSpec((MB, D), lambda m: (m, 0)),
            pl.BlockSpec((MB, D), lambda m: (m, 0)),
            cs((D, D)), cs((D, D)),
            cs((NC,) + DEG2),
        ],
        out_specs=[
            pl.BlockSpec((2, MB, D), lambda m: (0, m, 0)),
            cs(DEG2),
        ],
        out_shape=[
            jax.ShapeDtypeStruct((2, NP, D), jnp.bfloat16),
            jax.ShapeDtypeStruct(DEG2, jnp.float32),
        ],
    )(net_feat, dag_feat, net_W, dag_W, deg.reshape((NC,) + DEG2))


def _scmsg_body(einfo, xwi, dinv, z2, acc_out,
                eb0, eb1, ib0, ib1, cb0, cb1, rows0, rows1, nbuf, dinv_full,
                acc_sp, sem_e0, sem_e1, sem_g0, sem_g1, sem_s0, sem_s1):
    c = lax.axis_index("c")
    s = lax.axis_index("s")
    sl = pl.ds(s * SL2, SL2)
    pltpu.sync_copy(z2.at[sl], acc_sp.at[sl])
    pltpu.sync_copy(dinv, dinv_full)

    base = s * NBT
    pltpu.async_copy(einfo.at[base], eb0, sem_e0)
    pltpu.async_copy(einfo.at[base + 1], eb1, sem_e1)
    plsc.subcore_barrier()

    bufs = ((eb0, ib0, cb0, rows0, sem_e0, sem_g0, sem_s0),
            (eb1, ib1, cb1, rows1, sem_e1, sem_g1, sem_s1))

    def body(g, _):
        plsc.subcore_barrier()
        for b, (eb, ib, cb, rows, sem_e, sem_g, sem_s) in enumerate(bufs):
            i = g * 2 + b
            pltpu.make_async_copy(einfo.at[0], eb, sem_e).wait()

            @plsc.parallel_loop(0, K // L, unroll=4)
            def _(j):
                jj = pl.ds(j * L, L)
                ib[jj] = eb[0, jj] * 2 + c
                cb[jj] = eb[1, jj]

            @pl.when(g > 0)
            def _():
                pltpu.make_async_copy(
                    xwi.at[pl.ds(0, K)], rows, sem_s).wait()

            pltpu.async_copy(xwi.at[ib], rows, sem_g)
            @plsc.parallel_loop(0, K // L, unroll=4)
            def _(j):
                jj = pl.ds(j * L, L)
                r16 = eb[0, jj]
                ew16 = plsc.bitcast(eb[2, jj], jnp.float32)
                nbuf[jj] = ew16 * plsc.load_gather(dinv_full, [r16])

            @pl.when(i + 2 < NBT)
            def _():
                pltpu.async_copy(einfo.at[base + i + 2], eb, sem_e)

            pltpu.make_async_copy(xwi.at[pl.ds(0, K)], rows, sem_g).wait()

            @plsc.parallel_loop(0, K, unroll=16)
            def _(r):
                ns = plsc.load_gather(nbuf, [jnp.full((L,), r, jnp.int32)])
                nsb = plsc.pack(ns, ns, format=plsc.PackFormat.INTERLEAVED)
                for cc in range(H // 32):
                    rows[r, cc * 32:(cc + 1) * 32] = (
                        rows[r, cc * 32:(cc + 1) * 32] * nsb)

            pltpu.async_copy(rows, acc_sp.at[cb], sem_s, add=True)
        return 0

    lax.fori_loop(0, NBT // 2, body, 0)
    pltpu.make_async_copy(xwi.at[pl.ds(0, K)], rows0, sem_s0).wait()
    pltpu.make_async_copy(xwi.at[pl.ds(0, K)], rows1, sem_s1).wait()
    plsc.subcore_barrier()

    pltpu.sync_copy(acc_sp.at[sl], acc_out.at[c, sl])


def _sc_msg(einfo, xwi, dinv, z2):
    f32 = jnp.float32
    bf16 = jnp.bfloat16
    i32 = jnp.int32
    kern = pl.kernel(
        _scmsg_body,
        mesh=plsc.VectorSubcoreMesh(core_axis_name="c", subcore_axis_name="s"),
        compiler_params=pltpu.CompilerParams(
            needs_layout_passes=False, use_tc_tiling_on_sc=False),
        out_type=jax.ShapeDtypeStruct((NC, NN, H), bf16),
        scratch_types=(
            [pltpu.VMEM((3, K), i32)] * 2
            + [pltpu.VMEM((K,), i32)] * 4
            + [pltpu.VMEM((K, H), bf16)] * 2
            + [pltpu.VMEM((K,), f32), pltpu.VMEM((NN,), f32),
               pltpu.VMEM_SHARED((NN, H), bf16)]
            + [pltpu.SemaphoreType.DMA] * 6
        ),
    )
    return kern(einfo, xwi, dinv, z2)


def _t2_body(acc, dinv, bsel, act, A1, b1, A2, b2, F1, fb1, F2, fb2,
             out, s_ref):
    m = pl.program_id(0)
    nblk = pl.num_programs(0)
    BLK = acc.shape[1]

    @pl.when(m == 0)
    def _():
        s_ref[...] = jnp.zeros_like(s_ref)

    r = m * BLK + lax.broadcasted_iota(jnp.int32, (BLK, 1), 0)
    mask = ((r < N) | ((r >= NP) & (r < NP + N))).astype(jnp.float32)
    g = m // (nblk // 2)

    dv = dinv[...]
    bg = bsel[pl.ds(g, 1), :]
    a0 = acc[0].astype(jnp.float32)
    a1 = acc[1].astype(jnp.float32)
    v0 = jax.nn.relu(dv * a0 + bg[0:1, 0:H])
    v1 = jax.nn.relu(dv * a1 + bg[0:1, H:D])
    s0 = jnp.sum(v0 * mask, axis=0, keepdims=True)
    s1 = jnp.sum(v1 * mask, axis=0, keepdims=True)
    s_ref[pl.ds(g, 1), 0:H] += s0
    s_ref[pl.ds(g, 1), H:D] += s1

    @pl.when(m == nblk - 1)
    def _():
        inv_n = jnp.float32(1.0 / N)
        emb_n = s_ref[0:1, :] * inv_n
        emb_d = s_ref[1:2, :] * inv_n
        hh = act[...] @ A1[...] + b1[...]
        hh = hh * jnp.tanh(jax.nn.softplus(hh))
        ae = hh @ A2[...] + b2[...]
        h2 = jax.nn.relu(
            emb_n @ F1[0:D, :] + emb_d @ F1[D:2 * D, :]
            + ae @ F1[2 * D:3 * D, :] + fb1[...])
        sv = jnp.sum(h2 * F2[...].T, axis=1, keepdims=True) + fb2[...]
        out[...] = jnp.broadcast_to(sv, out.shape)


def _t2(acc, dinv, bsel, act, A1, b1, A2, b2, F1, fb1, F2, fb2):
    BLK = 512
    nblk = NN // BLK
    cs = lambda shape: pl.BlockSpec(shape, lambda m: tuple(0 for _ in shape))
    return pl.pallas_call(
        _t2_body,
        grid=(nblk,),
        in_specs=[
            pl.BlockSpec((NC, BLK, H), lambda m: (0, m, 0)),
            pl.BlockSpec((BLK, 1), lambda m: (m, 0)),
            cs((2, D)),
            cs((1, 512)),
            cs((512, D)), cs((1, D)),
            cs((D, D)), cs((1, D)),
            cs((3 * D, D)), cs((1, D)),
            cs((D, 1)), cs((1, 1)),
        ],
        out_specs=pl.BlockSpec((1, D), lambda m: (0, 0)),
        out_shape=jax.ShapeDtypeStruct((1, D), jnp.float32),
        scratch_shapes=[pltpu.VMEM((8, D), jnp.float32)],
    )(acc, dinv, bsel, act, A1, b1, A2, b2, F1, fb1, F2, fb2)


def _prep(nei, new, dei, dew):
    i32 = jnp.int32
    f32 = jnp.float32
    ar = jnp.arange(N, dtype=i32)
    pad = PE - (new.shape[0] + dew.shape[0] + 2 * N)
    row = jnp.concatenate(
        [nei[0], ar, NP + dei[0], NP + ar, jnp.zeros((pad,), i32)])
    col = jnp.concatenate(
        [nei[1], ar, NP + dei[1], NP + ar, jnp.full((pad,), N, i32)])
    ew = jnp.concatenate(
        [new, jnp.ones((N,), f32), dew, jnp.ones((N,), f32),
         jnp.zeros((pad,), f32)])
    einfo = jnp.stack([row, col, lax.bitcast_convert_type(ew, i32)])
    return einfo.reshape(3, NB, K).transpose(1, 0, 2)


def kernel(net_feat, net_edge_index, net_edge_weights, dag_feat,
           dag_edge_index, dag_edge_weights, action, net_W, net_b, dag_W,
           dag_b, A1, b1, A2, b2, F1, fb1, F2, fb2):
    einfo = _prep(net_edge_index, net_edge_weights,
                  dag_edge_index, dag_edge_weights)
    z1 = jnp.zeros((NN,), jnp.float32)
    z2 = jnp.zeros((NN, H), jnp.bfloat16)
    deg = _sc_deg(einfo, z1)
    pad_rows = jnp.zeros((NP - N, D), jnp.float32)
    nf = jnp.concatenate([net_feat, pad_rows])
    df = jnp.concatenate([dag_feat, pad_rows])
    xw, dinv = _t1(nf, net_W, df, dag_W, deg)
    acc = _sc_msg(einfo, xw.reshape(4 * NP, H), dinv.reshape(NN), z2)
    bsel = jnp.stack([net_b, dag_b])
    sv = _t2(acc, dinv.reshape(NN, 1), bsel, action.reshape(1, -1),
             A1, b1.reshape(1, D), A2, b2.reshape(1, D),
             F1, fb1.reshape(1, D), F2, fb2.reshape(1, 1))
    return sv[0, :1]


# speedup vs baseline: 1.0916x; 1.0916x over previous
import jax
import jax.numpy as jnp
from jax import lax
from jax.experimental import pallas as pl
from jax.experimental.pallas import tpu as pltpu
from jax.experimental.pallas import tpu_sc as plsc

N = 10000
D = 128
H = 64
NP = 10240
NN = 2 * NP
NC, NS, L = 2, 16, 16
K = 128
NB = 4032
PE = NB * K
NBT = NB // NS
NBC = NB // (NS * NC)
SL2 = NN // NS
DEG2 = (NN // D, D)


def _scdeg_body(einfo, z1, deg_out,
                eb0, eb1, cb0, cb1, ewb0, ewb1, deg_sp,
                sem_e0, sem_e1, sem_s0, sem_s1):
    c = lax.axis_index("c")
    s = lax.axis_index("s")
    sl = pl.ds(s * SL2, SL2)
    pltpu.sync_copy(z1.at[sl], deg_sp.at[sl])
    plsc.subcore_barrier()

    base = (c * NS + s) * NBC
    pltpu.async_copy(einfo.at[base], eb0, sem_e0)
    pltpu.async_copy(einfo.at[base + 1], eb1, sem_e1)

    bufs = ((eb0, cb0, ewb0, sem_e0, sem_s0),
            (eb1, cb1, ewb1, sem_e1, sem_s1))

    def body(g, _):
        for b, (eb, cb, ewb, sem_e, sem_s) in enumerate(bufs):
            i = g * 2 + b
            pltpu.make_async_copy(einfo.at[0], eb, sem_e).wait()

            @pl.when(g > 0)
            def _():
                pltpu.make_async_copy(z1.at[pl.ds(0, K)], ewb, sem_s).wait()

            for j in range(K // L):
                jj = pl.ds(j * L, L)
                cb[jj] = eb[1, jj]
                ewb[jj] = plsc.bitcast(eb[2, jj], jnp.float32)

            @pl.when(i + 2 < NBC)
            def _():
                pltpu.async_copy(einfo.at[base + i + 2], eb, sem_e)

            pltpu.async_copy(ewb, deg_sp.at[cb], sem_s, add=True)
        return 0

    lax.fori_loop(0, NBC // 2, body, 0)
    pltpu.make_async_copy(z1.at[pl.ds(0, K)], ewb0, sem_s0).wait()
    pltpu.make_async_copy(z1.at[pl.ds(0, K)], ewb1, sem_s1).wait()
    plsc.subcore_barrier()
    pltpu.sync_copy(deg_sp.at[sl], deg_out.at[c, sl])


def _sc_deg(einfo, z1):
    f32 = jnp.float32
    i32 = jnp.int32
    kern = pl.kernel(
        _scdeg_body,
        mesh=plsc.VectorSubcoreMesh(core_axis_name="c", subcore_axis_name="s"),
        compiler_params=pltpu.CompilerParams(
            needs_layout_passes=False, use_tc_tiling_on_sc=False),
        out_type=jax.ShapeDtypeStruct((NC, NN), f32),
        scratch_types=[
            pltpu.VMEM((3, K), i32), pltpu.VMEM((3, K), i32),
            pltpu.VMEM((K,), i32), pltpu.VMEM((K,), i32),
            pltpu.VMEM((K,), f32), pltpu.VMEM((K,), f32),
            pltpu.VMEM_SHARED((NN,), f32),
            pltpu.SemaphoreType.DMA, pltpu.SemaphoreType.DMA,
            pltpu.SemaphoreType.DMA, pltpu.SemaphoreType.DMA,
        ],
    )
    return kern(einfo, z1)


def _mm_body(nf, df, nw, dw, dg, xw, dv):
    xw[0] = jnp.dot(
        nf[...], nw[...], preferred_element_type=jnp.float32
    ).astype(jnp.bfloat16)
    xw[1] = jnp.dot(
        df[...], dw[...], preferred_element_type=jnp.float32
    ).astype(jnp.bfloat16)

    @pl.when(pl.program_id(0) == 0)
    def _():
        degsum = dg[0] + dg[1]
        dv[...] = jnp.where(degsum > 0, lax.rsqrt(degsum), 0.0)


def _t1(net_feat, net_W, dag_feat, dag_W, deg):
    MB = 640
    cs = lambda shape: pl.BlockSpec(shape, lambda m: tuple(0 for _ in shape))
    return pl.pallas_call(
        _mm_body,
        grid=(NP // MB,),
        in_specs=[
            pl.BlockSpec((MB, D), lambda m: (m, 0)),
            pl.BlockSpec((MB, D), lambda m: (m, 0)),
            cs((D, D)), cs((D, D)),
            cs((NC,) + DEG2),
        ],
        out_specs=[
            pl.BlockSpec((2, MB, D), lambda m: (0, m, 0)),
            cs(DEG2),
        ],
        out_shape=[
            jax.ShapeDtypeStruct((2, NP, D), jnp.bfloat16),
            jax.ShapeDtypeStruct(DEG2, jnp.float32),
        ],
    )(net_feat, dag_feat, net_W, dag_W, deg.reshape((NC,) + DEG2))


def _scmsg_body(einfo, xwi, dinv, z2, acc_out,
                eb0, eb1, eb2, ib0, ib1, ib2, cb0, cb1, cb2,
                rows0, rows1, rows2, nbuf, dinv_full, acc_sp,
                sem_e0, sem_e1, sem_e2, sem_g0, sem_g1, sem_g2,
                sem_s0, sem_s1, sem_s2):
    c = lax.axis_index("c")
    s = lax.axis_index("s")
    sl = pl.ds(s * SL2, SL2)
    pltpu.sync_copy(z2.at[sl], acc_sp.at[sl])
    pltpu.sync_copy(dinv, dinv_full)

    base = s * NBT
    pltpu.async_copy(einfo.at[base], eb0, sem_e0)
    pltpu.async_copy(einfo.at[base + 1], eb1, sem_e1)
    plsc.subcore_barrier()

    ebs = (eb0, eb1, eb2)
    ibs = (ib0, ib1, ib2)
    cbs = (cb0, cb1, cb2)
    rowss = (rows0, rows1, rows2)
    sem_es = (sem_e0, sem_e1, sem_e2)
    sem_gs = (sem_g0, sem_g1, sem_g2)
    sem_ss = (sem_s0, sem_s1, sem_s2)

    def build_and_gather(x):
        eb, ib, cb = ebs[x], ibs[x], cbs[x]

        @plsc.parallel_loop(0, K // L, unroll=4)
        def _(j):
            jj = pl.ds(j * L, L)
            ib[jj] = eb[0, jj] * 2 + c
            cb[jj] = eb[1, jj]

        pltpu.async_copy(xwi.at[ib], rowss[x], sem_gs[x])

    pltpu.make_async_copy(einfo.at[0], eb0, sem_e0).wait()
    build_and_gather(0)

    def body(g, _):
        plsc.subcore_barrier()
        for st in range(3):
            i = g * 3 + st
            x = st
            y = (st + 1) % 3
            z = (st + 2) % 3
            eb, rows = ebs[x], rowss[x]

            @plsc.parallel_loop(0, K // L, unroll=4)
            def _(j):
                jj = pl.ds(j * L, L)
                r16 = eb[0, jj]
                ew16 = plsc.bitcast(eb[2, jj], jnp.float32)
                nbuf[jj] = ew16 * plsc.load_gather(dinv_full, [r16])

            @pl.when(i + 1 < NBT)
            def _():
                pltpu.make_async_copy(einfo.at[0], ebs[y], sem_es[y]).wait()

                @pl.when(i >= 2)
                def _():
                    pltpu.make_async_copy(
                        xwi.at[pl.ds(0, K)], rowss[y], sem_ss[y]).wait()

                build_and_gather(y)

            @pl.when(i + 2 < NBT)
            def _():
                pltpu.async_copy(einfo.at[base + i + 2], ebs[z], sem_es[z])

            pltpu.make_async_copy(xwi.at[pl.ds(0, K)], rows, sem_gs[x]).wait()

            @plsc.parallel_loop(0, K, unroll=8)
            def _(r):
                ns = plsc.load_gather(nbuf, [jnp.full((L,), r, jnp.int32)])
                nsb = plsc.pack(ns, ns, format=plsc.PackFormat.INTERLEAVED)
                for cc in range(H // 32):
                    rows[r, cc * 32:(cc + 1) * 32] = (
                        rows[r, cc * 32:(cc + 1) * 32] * nsb)

            pltpu.async_copy(rows, acc_sp.at[cbs[x]], sem_ss[x], add=True)
        return 0

    lax.fori_loop(0, NBT // 3, body, 0)
    pltpu.make_async_copy(xwi.at[pl.ds(0, K)], rows0, sem_s0).wait()
    pltpu.make_async_copy(xwi.at[pl.ds(0, K)], rows1, sem_s1).wait()
    pltpu.make_async_copy(xwi.at[pl.ds(0, K)], rows2, sem_s2).wait()
    plsc.subcore_barrier()

    pltpu.sync_copy(acc_sp.at[sl], acc_out.at[c, sl])


def _sc_msg(einfo, xwi, dinv, z2):
    f32 = jnp.float32
    bf16 = jnp.bfloat16
    i32 = jnp.int32
    kern = pl.kernel(
        _scmsg_body,
        mesh=plsc.VectorSubcoreMesh(core_axis_name="c", subcore_axis_name="s"),
        compiler_params=pltpu.CompilerParams(
            needs_layout_passes=False, use_tc_tiling_on_sc=False),
        out_type=jax.ShapeDtypeStruct((NC, NN, H), bf16),
        scratch_types=(
            [pltpu.VMEM((3, K), i32)] * 3
            + [pltpu.VMEM((K,), i32)] * 6
            + [pltpu.VMEM((K, H), bf16)] * 3
            + [pltpu.VMEM((K,), f32), pltpu.VMEM((NN,), f32),
               pltpu.VMEM_SHARED((NN, H), bf16)]
            + [pltpu.SemaphoreType.DMA] * 9
        ),
    )
    return kern(einfo, xwi, dinv, z2)


def _t2_body(acc, dinv, bsel, act, A1, b1, A2, b2, F1, fb1, F2, fb2,
             out, s_ref):
    m = pl.program_id(0)
    nblk = pl.num_programs(0)
    BLK = acc.shape[1]

    @pl.when(m == 0)
    def _():
        s_ref[...] = jnp.zeros_like(s_ref)

    r = m * BLK + lax.broadcasted_iota(jnp.int32, (BLK, 1), 0)
    mask = ((r < N) | ((r >= NP) & (r < NP + N))).astype(jnp.float32)
    g = m // (nblk // 2)

    dv = dinv[...]
    bg = bsel[pl.ds(g, 1), :]
    a0 = acc[0].astype(jnp.float32)
    a1 = acc[1].astype(jnp.float32)
    v0 = jax.nn.relu(dv * a0 + bg[0:1, 0:H])
    v1 = jax.nn.relu(dv * a1 + bg[0:1, H:D])
    s0 = jnp.sum(v0 * mask, axis=0, keepdims=True)
    s1 = jnp.sum(v1 * mask, axis=0, keepdims=True)
    s_ref[pl.ds(g, 1), 0:H] += s0
    s_ref[pl.ds(g, 1), H:D] += s1

    @pl.when(m == nblk - 1)
    def _():
        inv_n = jnp.float32(1.0 / N)
        emb_n = s_ref[0:1, :] * inv_n
        emb_d = s_ref[1:2, :] * inv_n
        hh = act[...] @ A1[...] + b1[...]
        hh = hh * jnp.tanh(jax.nn.softplus(hh))
        ae = hh @ A2[...] + b2[...]
        h2 = jax.nn.relu(
            emb_n @ F1[0:D, :] + emb_d @ F1[D:2 * D, :]
            + ae @ F1[2 * D:3 * D, :] + fb1[...])
        sv = jnp.sum(h2 * F2[...].T, axis=1, keepdims=True) + fb2[...]
        out[...] = jnp.broadcast_to(sv, out.shape)


def _t2(acc, dinv, bsel, act, A1, b1, A2, b2, F1, fb1, F2, fb2):
    BLK = 512
    nblk = NN // BLK
    cs = lambda shape: pl.BlockSpec(shape, lambda m: tuple(0 for _ in shape))
    return pl.pallas_call(
        _t2_body,
        grid=(nblk,),
        in_specs=[
            pl.BlockSpec((NC, BLK, H), lambda m: (0, m, 0)),
            pl.BlockSpec((BLK, 1), lambda m: (m, 0)),
            cs((2, D)),
            cs((1, 512)),
            cs((512, D)), cs((1, D)),
            cs((D, D)), cs((1, D)),
            cs((3 * D, D)), cs((1, D)),
            cs((D, 1)), cs((1, 1)),
        ],
        out_specs=pl.BlockSpec((1, D), lambda m: (0, 0)),
        out_shape=jax.ShapeDtypeStruct((1, D), jnp.float32),
        scratch_shapes=[pltpu.VMEM((8, D), jnp.float32)],
    )(acc, dinv, bsel, act, A1, b1, A2, b2, F1, fb1, F2, fb2)


def _prep(nei, new, dei, dew):
    i32 = jnp.int32
    f32 = jnp.float32
    ar = jnp.arange(N, dtype=i32)
    pad = PE - (new.shape[0] + dew.shape[0] + 2 * N)
    row = jnp.concatenate(
        [nei[0], ar, NP + dei[0], NP + ar, jnp.zeros((pad,), i32)])
    col = jnp.concatenate(
        [nei[1], ar, NP + dei[1], NP + ar, jnp.full((pad,), N, i32)])
    ew = jnp.concatenate(
        [new, jnp.ones((N,), f32), dew, jnp.ones((N,), f32),
         jnp.zeros((pad,), f32)])
    einfo = jnp.stack([row, col, lax.bitcast_convert_type(ew, i32)])
    return einfo.reshape(3, NB, K).transpose(1, 0, 2)


def kernel(net_feat, net_edge_index, net_edge_weights, dag_feat,
           dag_edge_index, dag_edge_weights, action, net_W, net_b, dag_W,
           dag_b, A1, b1, A2, b2, F1, fb1, F2, fb2):
    einfo = _prep(net_edge_index, net_edge_weights,
                  dag_edge_index, dag_edge_weights)
    z1 = jnp.zeros((NN,), jnp.float32)
    z2 = jnp.zeros((NN, H), jnp.bfloat16)
    deg = _sc_deg(einfo, z1)
    pad_rows = jnp.zeros((NP - N, D), jnp.float32)
    nf = jnp.concatenate([net_feat, pad_rows])
    df = jnp.concatenate([dag_feat, pad_rows])
    xw, dinv = _t1(nf, net_W, df, dag_W, deg)
    acc = _sc_msg(einfo, xw.reshape(4 * NP, H), dinv.reshape(NN), z2)
    bsel = jnp.stack([net_b, dag_b])
    sv = _t2(acc, dinv.reshape(NN, 1), bsel, action.reshape(1, -1),
             A1, b1.reshape(1, D), A2, b2.reshape(1, D),
             F1, fb1.reshape(1, D), F2, fb2.reshape(1, 1))
    return sv[0, :1]


# speedup vs baseline: 1.1063x; 1.0134x over previous
import jax
import jax.numpy as jnp
from jax import lax
from jax.experimental import pallas as pl
from jax.experimental.pallas import tpu as pltpu
from jax.experimental.pallas import tpu_sc as plsc

N = 10000
D = 128
H = 64
NP = 10240
NN = 2 * NP
NC, NS, L = 2, 16, 16
K = 128
NB = 4032
PE = NB * K
NBT = NB // NS
NBC = NB // (NS * NC)
SL2 = NN // NS
DEG2 = (NN // D, D)


def _scdeg_body(einfo, z1, deg_out,
                eb0, eb1, cb0, cb1, ewb0, ewb1, deg_sp,
                sem_e0, sem_e1, sem_s0, sem_s1):
    c = lax.axis_index("c")
    s = lax.axis_index("s")
    sl = pl.ds(s * SL2, SL2)
    pltpu.sync_copy(z1.at[sl], deg_sp.at[sl])
    plsc.subcore_barrier()

    base = (c * NS + s) * NBC
    pltpu.async_copy(einfo.at[base], eb0, sem_e0)
    pltpu.async_copy(einfo.at[base + 1], eb1, sem_e1)

    bufs = ((eb0, cb0, ewb0, sem_e0, sem_s0),
            (eb1, cb1, ewb1, sem_e1, sem_s1))

    def body(g, _):
        for b, (eb, cb, ewb, sem_e, sem_s) in enumerate(bufs):
            i = g * 2 + b
            pltpu.make_async_copy(einfo.at[0], eb, sem_e).wait()

            @pl.when(g > 0)
            def _():
                pltpu.make_async_copy(z1.at[pl.ds(0, K)], ewb, sem_s).wait()

            @plsc.parallel_loop(0, K // L, unroll=4)
            def _(j):
                jj = pl.ds(j * L, L)
                cb[jj] = eb[1, jj]
                ewb[jj] = plsc.bitcast(eb[2, jj], jnp.float32)

            @pl.when(i + 2 < NBC)
            def _():
                pltpu.async_copy(einfo.at[base + i + 2], eb, sem_e)

            pltpu.async_copy(ewb, deg_sp.at[cb], sem_s, add=True)
        return 0

    lax.fori_loop(0, NBC // 2, body, 0)
    pltpu.make_async_copy(z1.at[pl.ds(0, K)], ewb0, sem_s0).wait()
    pltpu.make_async_copy(z1.at[pl.ds(0, K)], ewb1, sem_s1).wait()
    plsc.subcore_barrier()
    pltpu.sync_copy(deg_sp.at[sl], deg_out.at[c, sl])


def _sc_deg(einfo, z1):
    f32 = jnp.float32
    i32 = jnp.int32
    kern = pl.kernel(
        _scdeg_body,
        mesh=plsc.VectorSubcoreMesh(core_axis_name="c", subcore_axis_name="s"),
        compiler_params=pltpu.CompilerParams(
            needs_layout_passes=False, use_tc_tiling_on_sc=False),
        out_type=jax.ShapeDtypeStruct((NC, NN), f32),
        scratch_types=[
            pltpu.VMEM((3, K), i32), pltpu.VMEM((3, K), i32),
            pltpu.VMEM((K,), i32), pltpu.VMEM((K,), i32),
            pltpu.VMEM((K,), f32), pltpu.VMEM((K,), f32),
            pltpu.VMEM_SHARED((NN,), f32),
            pltpu.SemaphoreType.DMA, pltpu.SemaphoreType.DMA,
            pltpu.SemaphoreType.DMA, pltpu.SemaphoreType.DMA,
        ],
    )
    return kern(einfo, z1)


def _mm_body(nf, df, nw, dw, dg, xw, dv):
    xw[0] = jnp.dot(
        nf[...], nw[...], preferred_element_type=jnp.float32
    ).astype(jnp.bfloat16)
    xw[1] = jnp.dot(
        df[...], dw[...], preferred_element_type=jnp.float32
    ).astype(jnp.bfloat16)

    @pl.when(pl.program_id(0) == 0)
    def _():
        degsum = dg[0] + dg[1]
        dv[...] = jnp.where(degsum > 0, lax.rsqrt(degsum), 0.0)


def _t1(net_feat, net_W, dag_feat, dag_W, deg):
    MB = 640
    cs = lambda shape: pl.BlockSpec(shape, lambda m: tuple(0 for _ in shape))
    return pl.pallas_call(
        _mm_body,
        grid=(NP // MB,),
        in_specs=[
            pl.BlockSpec((MB, D), lambda m: (m, 0)),
            pl.BlockSpec((MB, D), lambda m: (m, 0)),
            cs((D, D)), cs((D, D)),
            cs((NC,) + DEG2),
        ],
        out_specs=[
            pl.BlockSpec((2, MB, D), lambda m: (0, m, 0)),
            cs(DEG2),
        ],
        out_shape=[
            jax.ShapeDtypeStruct((2, NP, D), jnp.bfloat16),
            jax.ShapeDtypeStruct(DEG2, jnp.float32),
        ],
    )(net_feat, dag_feat, net_W, dag_W, deg.reshape((NC,) + DEG2))


def _scmsg_body(einfo, xwi, dinv, z2, acc_out,
                eb0, eb1, eb2, ib0, ib1, ib2, cb0, cb1, cb2,
                rows0, rows1, rows2, nbuf, dinv_full, acc_sp,
                sem_e0, sem_e1, sem_e2, sem_g0, sem_g1, sem_g2,
                sem_s0, sem_s1, sem_s2):
    c = lax.axis_index("c")
    s = lax.axis_index("s")
    sl = pl.ds(s * SL2, SL2)
    pltpu.sync_copy(z2.at[sl], acc_sp.at[sl])
    pltpu.sync_copy(dinv, dinv_full)

    base = s * NBT
    pltpu.async_copy(einfo.at[base], eb0, sem_e0)
    pltpu.async_copy(einfo.at[base + 1], eb1, sem_e1)
    plsc.subcore_barrier()

    ebs = (eb0, eb1, eb2)
    ibs = (ib0, ib1, ib2)
    cbs = (cb0, cb1, cb2)
    rowss = (rows0, rows1, rows2)
    sem_es = (sem_e0, sem_e1, sem_e2)
    sem_gs = (sem_g0, sem_g1, sem_g2)
    sem_ss = (sem_s0, sem_s1, sem_s2)

    def build_and_gather(x):
        eb, ib, cb = ebs[x], ibs[x], cbs[x]

        @plsc.parallel_loop(0, K // L, unroll=4)
        def _(j):
            jj = pl.ds(j * L, L)
            ib[jj] = eb[0, jj] * 2 + c
            cb[jj] = eb[1, jj]

        pltpu.async_copy(xwi.at[ib], rowss[x], sem_gs[x])

    pltpu.make_async_copy(einfo.at[0], eb0, sem_e0).wait()
    build_and_gather(0)

    def body(g, _):
        plsc.subcore_barrier()
        for st in range(3):
            i = g * 3 + st
            x = st
            y = (st + 1) % 3
            z = (st + 2) % 3
            eb, rows = ebs[x], rowss[x]

            @plsc.parallel_loop(0, K // L, unroll=4)
            def _(j):
                jj = pl.ds(j * L, L)
                r16 = eb[0, jj]
                ew16 = plsc.bitcast(eb[2, jj], jnp.float32)
                nbuf[jj] = ew16 * plsc.load_gather(dinv_full, [r16])

            @pl.when(i + 1 < NBT)
            def _():
                pltpu.make_async_copy(einfo.at[0], ebs[y], sem_es[y]).wait()

                @pl.when(i >= 2)
                def _():
                    pltpu.make_async_copy(
                        xwi.at[pl.ds(0, K)], rowss[y], sem_ss[y]).wait()

                build_and_gather(y)

            @pl.when(i + 2 < NBT)
            def _():
                pltpu.async_copy(einfo.at[base + i + 2], ebs[z], sem_es[z])

            pltpu.make_async_copy(xwi.at[pl.ds(0, K)], rows, sem_gs[x]).wait()

            @plsc.parallel_loop(0, K, unroll=8)
            def _(r):
                ns = plsc.load_gather(nbuf, [jnp.full((L,), r, jnp.int32)])
                nsb = plsc.pack(ns, ns, format=plsc.PackFormat.INTERLEAVED)
                for cc in range(H // 32):
                    rows[r, cc * 32:(cc + 1) * 32] = (
                        rows[r, cc * 32:(cc + 1) * 32] * nsb)

            pltpu.async_copy(rows, acc_sp.at[cbs[x]], sem_ss[x], add=True)
        return 0

    lax.fori_loop(0, NBT // 3, body, 0)
    pltpu.make_async_copy(xwi.at[pl.ds(0, K)], rows0, sem_s0).wait()
    pltpu.make_async_copy(xwi.at[pl.ds(0, K)], rows1, sem_s1).wait()
    pltpu.make_async_copy(xwi.at[pl.ds(0, K)], rows2, sem_s2).wait()
    plsc.subcore_barrier()

    pltpu.sync_copy(acc_sp.at[sl], acc_out.at[c, sl])


def _sc_msg(einfo, xwi, dinv, z2):
    f32 = jnp.float32
    bf16 = jnp.bfloat16
    i32 = jnp.int32
    kern = pl.kernel(
        _scmsg_body,
        mesh=plsc.VectorSubcoreMesh(core_axis_name="c", subcore_axis_name="s"),
        compiler_params=pltpu.CompilerParams(
            needs_layout_passes=False, use_tc_tiling_on_sc=False),
        out_type=jax.ShapeDtypeStruct((NC, NN, H), bf16),
        scratch_types=(
            [pltpu.VMEM((3, K), i32)] * 3
            + [pltpu.VMEM((K,), i32)] * 6
            + [pltpu.VMEM((K, H), bf16)] * 3
            + [pltpu.VMEM((K,), f32), pltpu.VMEM((NN,), f32),
               pltpu.VMEM_SHARED((NN, H), bf16)]
            + [pltpu.SemaphoreType.DMA] * 9
        ),
    )
    return kern(einfo, xwi, dinv, z2)


def _t2_body(acc, dinv, bsel, act, A1, b1, A2, b2, F1, fb1, F2, fb2,
             out, s_ref):
    m = pl.program_id(0)
    nblk = pl.num_programs(0)
    BLK = acc.shape[1]

    @pl.when(m == 0)
    def _():
        s_ref[...] = jnp.zeros_like(s_ref)

    r = m * BLK + lax.broadcasted_iota(jnp.int32, (BLK, 1), 0)
    mask = ((r < N) | ((r >= NP) & (r < NP + N))).astype(jnp.float32)
    g = m // (nblk // 2)

    dv = dinv[...]
    bg = bsel[pl.ds(g, 1), :]
    a0 = acc[0].astype(jnp.float32)
    a1 = acc[1].astype(jnp.float32)
    v0 = jax.nn.relu(dv * a0 + bg[0:1, 0:H])
    v1 = jax.nn.relu(dv * a1 + bg[0:1, H:D])
    s0 = jnp.sum(v0 * mask, axis=0, keepdims=True)
    s1 = jnp.sum(v1 * mask, axis=0, keepdims=True)
    s_ref[pl.ds(g, 1), 0:H] += s0
    s_ref[pl.ds(g, 1), H:D] += s1

    @pl.when(m == nblk - 1)
    def _():
        inv_n = jnp.float32(1.0 / N)
        emb_n = s_ref[0:1, :] * inv_n
        emb_d = s_ref[1:2, :] * inv_n
        hh = act[...] @ A1[...] + b1[...]
        hh = hh * jnp.tanh(jax.nn.softplus(hh))
        ae = hh @ A2[...] + b2[...]
        h2 = jax.nn.relu(
            emb_n @ F1[0:D, :] + emb_d @ F1[D:2 * D, :]
            + ae @ F1[2 * D:3 * D, :] + fb1[...])
        sv = jnp.sum(h2 * F2[...].T, axis=1, keepdims=True) + fb2[...]
        out[...] = jnp.broadcast_to(sv, out.shape)


def _t2(acc, dinv, bsel, act, A1, b1, A2, b2, F1, fb1, F2, fb2):
    BLK = 512
    nblk = NN // BLK
    cs = lambda shape: pl.BlockSpec(shape, lambda m: tuple(0 for _ in shape))
    return pl.pallas_call(
        _t2_body,
        grid=(nblk,),
        in_specs=[
            pl.BlockSpec((NC, BLK, H), lambda m: (0, m, 0)),
            pl.BlockSpec((BLK, 1), lambda m: (m, 0)),
            cs((2, D)),
            cs((1, 512)),
            cs((512, D)), cs((1, D)),
            cs((D, D)), cs((1, D)),
            cs((3 * D, D)), cs((1, D)),
            cs((D, 1)), cs((1, 1)),
        ],
        out_specs=pl.BlockSpec((1, D), lambda m: (0, 0)),
        out_shape=jax.ShapeDtypeStruct((1, D), jnp.float32),
        scratch_shapes=[pltpu.VMEM((8, D), jnp.float32)],
    )(acc, dinv, bsel, act, A1, b1, A2, b2, F1, fb1, F2, fb2)


def _prep(nei, new, dei, dew):
    i32 = jnp.int32
    f32 = jnp.float32
    ar = jnp.arange(N, dtype=i32)
    pad = PE - (new.shape[0] + dew.shape[0] + 2 * N)
    row = jnp.concatenate(
        [nei[0], ar, NP + dei[0], NP + ar, jnp.zeros((pad,), i32)])
    col = jnp.concatenate(
        [nei[1], ar, NP + dei[1], NP + ar, jnp.full((pad,), N, i32)])
    ew = jnp.concatenate(
        [new, jnp.ones((N,), f32), dew, jnp.ones((N,), f32),
         jnp.zeros((pad,), f32)])
    return jnp.concatenate(
        [row.reshape(NB, 1, K), col.reshape(NB, 1, K),
         lax.bitcast_convert_type(ew, i32).reshape(NB, 1, K)], axis=1)


def kernel(net_feat, net_edge_index, net_edge_weights, dag_feat,
           dag_edge_index, dag_edge_weights, action, net_W, net_b, dag_W,
           dag_b, A1, b1, A2, b2, F1, fb1, F2, fb2):
    einfo = _prep(net_edge_index, net_edge_weights,
                  dag_edge_index, dag_edge_weights)
    z1 = jnp.zeros((NN,), jnp.float32)
    z2 = jnp.zeros((NN, H), jnp.bfloat16)
    deg = _sc_deg(einfo, z1)
    pad_rows = jnp.zeros((NP - N, D), jnp.float32)
    nf = jnp.concatenate([net_feat, pad_rows])
    df = jnp.concatenate([dag_feat, pad_rows])
    xw, dinv = _t1(nf, net_W, df, dag_W, deg)
    acc = _sc_msg(einfo, xw.reshape(4 * NP, H), dinv.reshape(NN), z2)
    bsel = jnp.stack([net_b, dag_b])
    sv = _t2(acc, dinv.reshape(NN, 1), bsel, action.reshape(1, -1),
             A1, b1.reshape(1, D), A2, b2.reshape(1, D),
             F1, fb1.reshape(1, D), F2, fb2.reshape(1, 1))
    return sv[0, :1]
